# Initial kernel scaffold; baseline (speedup 1.0000x reference)
import jax
import jax.numpy as jnp
from jax.experimental import pallas as pl


def kernel(x, edge_index, edge_attr, W, b, alpha):
    raise NotImplementedError("write your pallas kernel here")


# speedup vs baseline: 8.7425x; 8.7425x over previous
import functools

import jax
import jax.numpy as jnp
from jax import lax
from jax.experimental import pallas as pl
from jax.experimental.pallas import tpu as pltpu
from jax.experimental.pallas import tpu_sc as plsc

NS = 16
CS = 2000
CG = 128


def _sc_pass_body(n_k, n_pad, e_total, n_nodes,
                  h_hbm, src_hbm, dst_hbm, attr_hbm, zeros_hbm, out_hbm,
                  stage_src, stage_dst, stage_attr, comp_src, comp_dst,
                  idx_src, idx_dst, rows, zsh, sem):
    c = lax.axis_index("c")
    s = lax.axis_index("s")
    epw = e_total // NS
    rpt = n_pad // NS
    base_e = s * epw
    n_groups = (n_k + 1) // 2

    for g in range(n_groups):
        k = 2 * g + c + 1
        active = k <= n_k

        @pl.when(active)
        def _zero():
            pltpu.sync_copy(zeros_hbm, zsh.at[pl.ds(s * rpt, rpt)])

        plsc.subcore_barrier()

        @pl.when(active)
        def _work():
            def gs_chunk(off):
                for j in range(CG // 16):
                    idx_src[pl.ds(j * 16, 16)] = comp_src[pl.ds(off + j * 16, 16)]
                    idx_dst[pl.ds(j * 16, 16)] = comp_dst[pl.ds(off + j * 16, 16)]
                pltpu.async_copy(h_hbm.at[idx_src], rows, sem).wait()
                pltpu.sync_copy(rows, zsh.at[idx_dst], add=True)

            def chunk_body(ci, tail):
                off = base_e + ci * CS
                pltpu.sync_copy(src_hbm.at[pl.ds(off, CS)], stage_src)
                pltpu.sync_copy(dst_hbm.at[pl.ds(off, CS)], stage_dst)
                pltpu.sync_copy(attr_hbm.at[pl.ds(off, CS)], stage_attr)

                def vec_body(v, n):
                    a16 = stage_attr[pl.ds(v * 16, 16)]
                    m = a16 == k
                    mi = m.astype(jnp.int32)
                    pos = n + plsc.cumsum(mi) - 1
                    plsc.store_scatter(comp_src, [pos],
                                       stage_src[pl.ds(v * 16, 16)], mask=m)
                    plsc.store_scatter(comp_dst, [pos],
                                       stage_dst[pl.ds(v * 16, 16)], mask=m)
                    return n + jnp.sum(mi)

                tail = lax.fori_loop(0, CS // 16, vec_body, tail)

                nfull = lax.div(tail, CG)

                def flush_body(i, carry):
                    gs_chunk(i * CG)
                    return carry

                lax.fori_loop(0, nfull, flush_body, jnp.int32(0))
                base = nfull * CG
                for j in range(CG // 16):
                    t_s = comp_src[pl.ds(base + j * 16, 16)]
                    t_d = comp_dst[pl.ds(base + j * 16, 16)]
                    comp_src[pl.ds(j * 16, 16)] = t_s
                    comp_dst[pl.ds(j * 16, 16)] = t_d
                return tail - base

            tail = lax.fori_loop(0, epw // CS, chunk_body, jnp.int32(0))

            iota16 = lax.iota(jnp.int32, 16)
            zero16 = jnp.zeros((16,), jnp.int32)
            dummy16 = jnp.full((16,), n_nodes, jnp.int32)
            for j in range(CG // 16):
                pos = tail + j * 16 + iota16
                plsc.store_scatter(comp_src, [pos], zero16)
                plsc.store_scatter(comp_dst, [pos], dummy16)

            def last_body(i, carry):
                gs_chunk(0)
                return carry

            lax.fori_loop(0, lax.div(tail + CG - 1, CG), last_body, jnp.int32(0))

        plsc.subcore_barrier()

        @pl.when(active)
        def _dump():
            pltpu.sync_copy(zsh.at[pl.ds(s * rpt, rpt)],
                            out_hbm.at[pl.ds((k - 1) * n_pad + s * rpt, rpt)])

        plsc.subcore_barrier()


@functools.lru_cache(maxsize=None)
def _make_sc_pass(n_k, n_pad, e_total, n_nodes, d):
    epw = e_total // NS
    mesh = plsc.VectorSubcoreMesh(core_axis_name="c", subcore_axis_name="s")
    return pl.kernel(
        functools.partial(_sc_pass_body, n_k, n_pad, e_total, n_nodes),
        out_type=jax.ShapeDtypeStruct((n_k * n_pad, d), jnp.float32),
        mesh=mesh,
        compiler_params=pltpu.CompilerParams(needs_layout_passes=False),
        scratch_types=[
            pltpu.VMEM((CS,), jnp.int32),
            pltpu.VMEM((CS,), jnp.int32),
            pltpu.VMEM((CS,), jnp.int32),
            pltpu.VMEM((CS + 2 * CG + 16,), jnp.int32),
            pltpu.VMEM((CS + 2 * CG + 16,), jnp.int32),
            pltpu.VMEM((CG,), jnp.int32),
            pltpu.VMEM((CG,), jnp.int32),
            pltpu.VMEM((CG, d), jnp.float32),
            pltpu.VMEM_SHARED((n_pad, d), jnp.float32),
            pltpu.SemaphoreType.DMA,
        ],
    )


def _tc_layer_body(t, alpha_ref, w_ref, b_ref, h_ref, *rest):
    z_refs = rest[:-1]
    out_ref = rest[-1]
    arow = alpha_ref[...]
    col = lax.broadcasted_iota(jnp.int32, arow.shape, 1)
    valid = col < (t + 1)
    masked = jnp.where(valid, arow, -1e30)
    mx = jnp.max(masked, axis=1, keepdims=True)
    ex = jnp.where(valid, jnp.exp(arow - mx), 0.0)
    denom = jnp.sum(ex, axis=1, keepdims=True)
    S = jnp.zeros_like(h_ref[...])
    for i in range(t + 1):
        wi = ex[0:1, i:i + 1] / denom
        S = S + wi * z_refs[i][...]
    acc = jnp.dot(S, w_ref[...], preferred_element_type=jnp.float32)
    acc = acc + b_ref[...]
    hn = h_ref[...] + jnp.maximum(acc, 0.0)
    nrm = jnp.sqrt(jnp.sum(hn * hn, axis=1, keepdims=True)) + 1e-12
    out_ref[...] = hn / nrm


@functools.lru_cache(maxsize=None)
def _make_tc_layer(t, n_nodes, d, n_alpha, bn):
    grid = (n_nodes // bn,)
    in_specs = [
        pl.BlockSpec((1, n_alpha), lambda i: (0, 0)),
        pl.BlockSpec((d, d), lambda i: (0, 0)),
        pl.BlockSpec((1, d), lambda i: (0, 0)),
        pl.BlockSpec((bn, d), lambda i: (i, 0)),
    ] + [pl.BlockSpec((bn, d), lambda i: (i, 0))] * (t + 1)
    return pl.pallas_call(
        functools.partial(_tc_layer_body, t),
        grid=grid,
        in_specs=in_specs,
        out_specs=pl.BlockSpec((bn, d), lambda i: (i, 0)),
        out_shape=jax.ShapeDtypeStruct((n_nodes, d), jnp.float32),
    )


def kernel(x, edge_index, edge_attr, W, b, alpha):
    n_nodes, d = x.shape
    num_layers = W.shape[0]
    e_total = edge_index.shape[1]
    n_pad = ((n_nodes + NS * 8 - 1) // (NS * 8)) * (NS * 8)

    src = edge_index[0].astype(jnp.int32)
    dst = edge_index[1].astype(jnp.int32)
    attr = edge_attr.astype(jnp.int32)
    zeros_blk = jnp.zeros((n_pad // NS, d), jnp.float32)
    alpha = alpha.astype(jnp.float32)

    h = x.astype(jnp.float32)
    Zs = []
    for t in range(num_layers):
        n_k = num_layers - t
        zflat = _make_sc_pass(n_k, n_pad, e_total, n_nodes, d)(
            h, src, dst, attr, zeros_blk)
        Zs.append(zflat.reshape(n_k, n_pad, d))
        zlist = [Zs[t - k + 1][k - 1, :n_nodes] for k in range(1, t + 2)]
        h = _make_tc_layer(t, n_nodes, d, alpha.shape[1], 1000)(
            alpha[t:t + 1], W[t], b[t].reshape(1, d), h, *zlist)
    return h


# speedup vs baseline: 9.2875x; 1.0623x over previous
import functools

import jax
import jax.numpy as jnp
from jax import lax
from jax.experimental import pallas as pl
from jax.experimental.pallas import tpu as pltpu
from jax.experimental.pallas import tpu_sc as plsc

NS = 16
CS = 2000
CG = 128


def _sc_pass_body(n_k, n_pad, e_total, n_nodes,
                  h_hbm, src_hbm, dst_hbm, attr_hbm, zeros_hbm, out_hbm,
                  stage_src, stage_dst, stage_attr, comp_src, comp_dst,
                  idx_src0, idx_dst0, idx_src1, idx_dst1, rows0, rows1,
                  zsh, semg, sems):
    c = lax.axis_index("c")
    s = lax.axis_index("s")
    epw = e_total // NS
    rpt = n_pad // NS
    base_e = s * epw
    n_groups = (n_k + 1) // 2

    for g in range(n_groups):
        k = 2 * g + c + 1
        active = k <= n_k

        @pl.when(active)
        def _zero():
            pltpu.sync_copy(zeros_hbm, zsh.at[pl.ds(s * rpt, rpt)])

        plsc.subcore_barrier()

        @pl.when(active)
        def _work():
            def fill_idx(bufs, bufd, off):
                for j in range(CG // 16):
                    bufs[pl.ds(j * 16, 16)] = comp_src[pl.ds(off + j * 16, 16)]
                    bufd[pl.ds(j * 16, 16)] = comp_dst[pl.ds(off + j * 16, 16)]

            def gs_range(nfull):
                def pair_body(p, carry):
                    i0 = 2 * p
                    both = i0 + 1 < nfull
                    fill_idx(idx_src0, idx_dst0, i0 * CG)
                    pltpu.async_copy(h_hbm.at[idx_src0], rows0, semg)

                    @pl.when(both)
                    def _():
                        fill_idx(idx_src1, idx_dst1, (i0 + 1) * CG)
                        pltpu.async_copy(h_hbm.at[idx_src1], rows1, semg)

                    pltpu.make_async_copy(h_hbm.at[idx_src0], rows0, semg).wait()

                    @pl.when(both)
                    def _():
                        pltpu.make_async_copy(h_hbm.at[idx_src1], rows1, semg).wait()

                    pltpu.async_copy(rows0, zsh.at[idx_dst0], sems, add=True)

                    @pl.when(both)
                    def _():
                        pltpu.async_copy(rows1, zsh.at[idx_dst1], sems, add=True)

                    pltpu.make_async_copy(rows0, zsh.at[idx_dst0], sems).wait()

                    @pl.when(both)
                    def _():
                        pltpu.make_async_copy(rows1, zsh.at[idx_dst1], sems).wait()

                    return carry

                lax.fori_loop(0, lax.div(nfull + 1, 2), pair_body, jnp.int32(0))

            def chunk_body(ci, tail):
                off = base_e + ci * CS
                pltpu.sync_copy(src_hbm.at[pl.ds(off, CS)], stage_src)
                pltpu.sync_copy(dst_hbm.at[pl.ds(off, CS)], stage_dst)
                pltpu.sync_copy(attr_hbm.at[pl.ds(off, CS)], stage_attr)

                def vec_body(v, n):
                    a16 = stage_attr[pl.ds(v * 16, 16)]
                    m = a16 == k
                    mi = m.astype(jnp.int32)
                    pos = n + plsc.cumsum(mi) - 1
                    plsc.store_scatter(comp_src, [pos],
                                       stage_src[pl.ds(v * 16, 16)], mask=m)
                    plsc.store_scatter(comp_dst, [pos],
                                       stage_dst[pl.ds(v * 16, 16)], mask=m)
                    return n + jnp.sum(mi)

                tail = lax.fori_loop(0, CS // 16, vec_body, tail)

                nfull = lax.div(tail, CG)
                gs_range(nfull)
                base = nfull * CG
                for j in range(CG // 16):
                    t_s = comp_src[pl.ds(base + j * 16, 16)]
                    t_d = comp_dst[pl.ds(base + j * 16, 16)]
                    comp_src[pl.ds(j * 16, 16)] = t_s
                    comp_dst[pl.ds(j * 16, 16)] = t_d
                return tail - base

            tail = lax.fori_loop(0, epw // CS, chunk_body, jnp.int32(0))

            iota16 = lax.iota(jnp.int32, 16)
            zero16 = jnp.zeros((16,), jnp.int32)
            dummy16 = jnp.full((16,), n_nodes, jnp.int32)
            for j in range(CG // 16):
                pos = tail + j * 16 + iota16
                plsc.store_scatter(comp_src, [pos], zero16)
                plsc.store_scatter(comp_dst, [pos], dummy16)

            gs_range(lax.div(tail + CG - 1, CG))

        plsc.subcore_barrier()

        @pl.when(active)
        def _dump():
            pltpu.sync_copy(zsh.at[pl.ds(s * rpt, rpt)],
                            out_hbm.at[pl.ds((k - 1) * n_pad + s * rpt, rpt)])

        plsc.subcore_barrier()


@functools.lru_cache(maxsize=None)
def _make_sc_pass(n_k, n_pad, e_total, n_nodes, d):
    epw = e_total // NS
    mesh = plsc.VectorSubcoreMesh(core_axis_name="c", subcore_axis_name="s")
    return pl.kernel(
        functools.partial(_sc_pass_body, n_k, n_pad, e_total, n_nodes),
        out_type=jax.ShapeDtypeStruct((n_k * n_pad, d), jnp.float32),
        mesh=mesh,
        compiler_params=pltpu.CompilerParams(needs_layout_passes=False),
        scratch_types=[
            pltpu.VMEM((CS,), jnp.int32),
            pltpu.VMEM((CS,), jnp.int32),
            pltpu.VMEM((CS,), jnp.int32),
            pltpu.VMEM((CS + 2 * CG + 16,), jnp.int32),
            pltpu.VMEM((CS + 2 * CG + 16,), jnp.int32),
            pltpu.VMEM((CG,), jnp.int32),
            pltpu.VMEM((CG,), jnp.int32),
            pltpu.VMEM((CG,), jnp.int32),
            pltpu.VMEM((CG,), jnp.int32),
            pltpu.VMEM((CG, d), jnp.float32),
            pltpu.VMEM((CG, d), jnp.float32),
            pltpu.VMEM_SHARED((n_pad, d), jnp.float32),
            pltpu.SemaphoreType.DMA,
            pltpu.SemaphoreType.DMA,
        ],
    )


def _tc_layer_body(t, alpha_ref, w_ref, b_ref, h_ref, *rest):
    z_refs = rest[:-1]
    out_ref = rest[-1]
    arow = alpha_ref[...]
    col = lax.broadcasted_iota(jnp.int32, arow.shape, 1)
    valid = col < (t + 1)
    masked = jnp.where(valid, arow, -1e30)
    mx = jnp.max(masked, axis=1, keepdims=True)
    ex = jnp.where(valid, jnp.exp(arow - mx), 0.0)
    denom = jnp.sum(ex, axis=1, keepdims=True)
    S = jnp.zeros_like(h_ref[...])
    for i in range(t + 1):
        wi = ex[0:1, i:i + 1] / denom
        S = S + wi * z_refs[i][...]
    acc = jnp.dot(S, w_ref[...], preferred_element_type=jnp.float32)
    acc = acc + b_ref[...]
    hn = h_ref[...] + jnp.maximum(acc, 0.0)
    nrm = jnp.sqrt(jnp.sum(hn * hn, axis=1, keepdims=True)) + 1e-12
    out_ref[...] = hn / nrm


@functools.lru_cache(maxsize=None)
def _make_tc_layer(t, n_nodes, d, n_alpha, bn):
    grid = (n_nodes // bn,)
    in_specs = [
        pl.BlockSpec((1, n_alpha), lambda i: (0, 0)),
        pl.BlockSpec((d, d), lambda i: (0, 0)),
        pl.BlockSpec((1, d), lambda i: (0, 0)),
        pl.BlockSpec((bn, d), lambda i: (i, 0)),
    ] + [pl.BlockSpec((bn, d), lambda i: (i, 0))] * (t + 1)
    return pl.pallas_call(
        functools.partial(_tc_layer_body, t),
        grid=grid,
        in_specs=in_specs,
        out_specs=pl.BlockSpec((bn, d), lambda i: (i, 0)),
        out_shape=jax.ShapeDtypeStruct((n_nodes, d), jnp.float32),
    )


def kernel(x, edge_index, edge_attr, W, b, alpha):
    n_nodes, d = x.shape
    num_layers = W.shape[0]
    e_total = edge_index.shape[1]
    n_pad = ((n_nodes + NS * 8 - 1) // (NS * 8)) * (NS * 8)

    src = edge_index[0].astype(jnp.int32)
    dst = edge_index[1].astype(jnp.int32)
    attr = edge_attr.astype(jnp.int32)
    zeros_blk = jnp.zeros((n_pad // NS, d), jnp.float32)
    alpha = alpha.astype(jnp.float32)

    h = x.astype(jnp.float32)
    Zs = []
    for t in range(num_layers):
        n_k = num_layers - t
        zflat = _make_sc_pass(n_k, n_pad, e_total, n_nodes, d)(
            h, src, dst, attr, zeros_blk)
        Zs.append(zflat.reshape(n_k, n_pad, d))
        zlist = [Zs[t - k + 1][k - 1, :n_nodes] for k in range(1, t + 2)]
        h = _make_tc_layer(t, n_nodes, d, alpha.shape[1], 1000)(
            alpha[t:t + 1], W[t], b[t].reshape(1, d), h, *zlist)
    return h


# speedup vs baseline: 11.0127x; 1.1858x over previous
import functools

import jax
import jax.numpy as jnp
from jax import lax
from jax.experimental import pallas as pl
from jax.experimental.pallas import tpu as pltpu
from jax.experimental.pallas import tpu_sc as plsc

NS = 16
CS = 2000
CG = 128
SB = 16
CAP = 20480


def _fill_idx(csrc, cdst, bufs, bufd, off):
    for j in range(CG // 16):
        bufs[pl.ds(j * 16, 16)] = csrc[pl.ds(off + j * 16, 16)]
        bufd[pl.ds(j * 16, 16)] = cdst[pl.ds(off + j * 16, 16)]


def _gs_range(h_hbm, zsh, csrc, cdst,
              idx_src0, idx_dst0, idx_src1, idx_dst1, rows0, rows1,
              semg, sems, nfull):
    def pair_body(p, carry):
        i0 = 2 * p
        both = i0 + 1 < nfull
        _fill_idx(csrc, cdst, idx_src0, idx_dst0, i0 * CG)
        pltpu.async_copy(h_hbm.at[idx_src0], rows0, semg)

        @pl.when(both)
        def _():
            _fill_idx(csrc, cdst, idx_src1, idx_dst1, (i0 + 1) * CG)
            pltpu.async_copy(h_hbm.at[idx_src1], rows1, semg)

        pltpu.make_async_copy(h_hbm.at[idx_src0], rows0, semg).wait()

        @pl.when(both)
        def _():
            pltpu.make_async_copy(h_hbm.at[idx_src1], rows1, semg).wait()

        pltpu.async_copy(rows0, zsh.at[idx_dst0], sems, add=True)

        @pl.when(both)
        def _():
            pltpu.async_copy(rows1, zsh.at[idx_dst1], sems, add=True)

        pltpu.make_async_copy(rows0, zsh.at[idx_dst0], sems).wait()

        @pl.when(both)
        def _():
            pltpu.make_async_copy(rows1, zsh.at[idx_dst1], sems).wait()

        return carry

    lax.fori_loop(0, lax.div(nfull + 1, 2), pair_body, jnp.int32(0))


def _list_gs(h_hbm, zsh, lsrc_hbm, ldst_hbm, ca_src, ca_dst,
             idx_src0, idx_dst0, idx_src1, idx_dst1, rows0, rows1,
             semg, sems, semst, seg, nchunks):
    def sbody(si, carry):
        off = seg * CAP + si * (SB * CG)
        pltpu.async_copy(lsrc_hbm.at[pl.ds(off, SB * CG)],
                         ca_src.at[pl.ds(0, SB * CG)], semst)
        pltpu.async_copy(ldst_hbm.at[pl.ds(off, SB * CG)],
                         ca_dst.at[pl.ds(0, SB * CG)], semst)
        pltpu.make_async_copy(lsrc_hbm.at[pl.ds(off, SB * CG)],
                              ca_src.at[pl.ds(0, SB * CG)], semst).wait()
        pltpu.make_async_copy(ldst_hbm.at[pl.ds(off, SB * CG)],
                              ca_dst.at[pl.ds(0, SB * CG)], semst).wait()
        _gs_range(h_hbm, zsh, ca_src, ca_dst,
                  idx_src0, idx_dst0, idx_src1, idx_dst1, rows0, rows1,
                  semg, sems, jnp.minimum(nchunks - si * SB, SB))
        return carry

    lax.fori_loop(0, lax.div(nchunks + SB - 1, SB), sbody, jnp.int32(0))


def _sc_pass0_body(n_pad, e_total, n_nodes,
                   h_hbm, src_hbm, dst_hbm, attr_hbm, zeros_hbm,
                   out_hbm, lsrc_hbm, ldst_hbm, cnt_hbm,
                   stage_src, stage_dst, stage_attr,
                   ca_src, ca_dst, cb_src, cb_dst,
                   idx_src0, idx_dst0, idx_src1, idx_dst1, rows0, rows1,
                   cnt16, zsh, semg, sems, semd, semst):
    c = lax.axis_index("c")
    s = lax.axis_index("s")
    epw = e_total // NS
    rpt = n_pad // NS
    base_e = s * epw
    k_a = c + 1
    k_b = c + 3
    seg_a = (k_a - 1) * NS + s
    seg_b = (k_b - 1) * NS + s

    def dump_chunks(csrc, cdst, seg, glob, nfull):
        def dbody(i, carry):
            o = seg * CAP + (glob + i) * CG
            pltpu.async_copy(csrc.at[pl.ds(i * CG, CG)], lsrc_hbm.at[pl.ds(o, CG)], semd)
            pltpu.async_copy(cdst.at[pl.ds(i * CG, CG)], ldst_hbm.at[pl.ds(o, CG)], semd)
            return carry

        lax.fori_loop(0, nfull, dbody, jnp.int32(0))

    def drain_dumps(n2):
        def dbody(i, carry):
            pltpu.make_async_copy(ca_src.at[pl.ds(0, CG)],
                                  lsrc_hbm.at[pl.ds(0, CG)], semd).wait()
            return carry

        lax.fori_loop(0, n2, dbody, jnp.int32(0))

    def slide(csrc, cdst, base):
        for j in range(CG // 16):
            t_s = csrc[pl.ds(base + j * 16, 16)]
            t_d = cdst[pl.ds(base + j * 16, 16)]
            csrc[pl.ds(j * 16, 16)] = t_s
            cdst[pl.ds(j * 16, 16)] = t_d

    def pad_tail(csrc, cdst, tail):
        iota16 = lax.iota(jnp.int32, 16)
        zero16 = jnp.zeros((16,), jnp.int32)
        dummy16 = jnp.full((16,), n_nodes, jnp.int32)
        for j in range(CG // 16):
            pos = tail + j * 16 + iota16
            plsc.store_scatter(csrc, [pos], zero16)
            plsc.store_scatter(cdst, [pos], dummy16)

    pltpu.sync_copy(zeros_hbm, zsh.at[pl.ds(s * rpt, rpt)])
    plsc.subcore_barrier()

    def chunk_body(ci, carry):
        tail_a, tail_b, glob_a, glob_b = carry
        off = base_e + ci * CS
        pltpu.async_copy(src_hbm.at[pl.ds(off, CS)], stage_src, semst)
        pltpu.async_copy(dst_hbm.at[pl.ds(off, CS)], stage_dst, semst)
        pltpu.async_copy(attr_hbm.at[pl.ds(off, CS)], stage_attr, semst)
        pltpu.make_async_copy(src_hbm.at[pl.ds(off, CS)], stage_src, semst).wait()
        pltpu.make_async_copy(dst_hbm.at[pl.ds(off, CS)], stage_dst, semst).wait()
        pltpu.make_async_copy(attr_hbm.at[pl.ds(off, CS)], stage_attr, semst).wait()

        def vec_body(v, carry2):
            na, nb = carry2
            a16 = stage_attr[pl.ds(v * 16, 16)]
            s16 = stage_src[pl.ds(v * 16, 16)]
            d16 = stage_dst[pl.ds(v * 16, 16)]
            ma = a16 == k_a
            mb = a16 == k_b
            posa = na + plsc.cumsum(ma.astype(jnp.int32)) - 1
            posb = nb + plsc.cumsum(mb.astype(jnp.int32)) - 1
            plsc.store_scatter(ca_src, [posa], s16, mask=ma)
            plsc.store_scatter(ca_dst, [posa], d16, mask=ma)
            plsc.store_scatter(cb_src, [posb], s16, mask=mb)
            plsc.store_scatter(cb_dst, [posb], d16, mask=mb)
            return posa[15] + 1, posb[15] + 1

        tail_a, tail_b = lax.fori_loop(0, CS // 16, vec_body, (tail_a, tail_b))

        nfa = lax.div(tail_a, CG)
        nfb = lax.div(tail_b, CG)
        dump_chunks(ca_src, ca_dst, seg_a, glob_a, nfa)
        dump_chunks(cb_src, cb_dst, seg_b, glob_b, nfb)
        _gs_range(h_hbm, zsh, ca_src, ca_dst,
                  idx_src0, idx_dst0, idx_src1, idx_dst1, rows0, rows1,
                  semg, sems, nfa)
        drain_dumps(2 * (nfa + nfb))
        slide(ca_src, ca_dst, nfa * CG)
        slide(cb_src, cb_dst, nfb * CG)
        return (tail_a - nfa * CG, tail_b - nfb * CG, glob_a + nfa, glob_b + nfb)

    z32 = jnp.int32(0)
    tail_a, tail_b, glob_a, glob_b = lax.fori_loop(
        0, epw // CS, chunk_body, (z32, z32, z32, z32))

    pad_tail(ca_src, ca_dst, tail_a)
    pad_tail(cb_src, cb_dst, tail_b)
    nla = lax.div(tail_a + CG - 1, CG)
    nlb = lax.div(tail_b + CG - 1, CG)
    dump_chunks(ca_src, ca_dst, seg_a, glob_a, nla)
    dump_chunks(cb_src, cb_dst, seg_b, glob_b, nlb)
    _gs_range(h_hbm, zsh, ca_src, ca_dst,
              idx_src0, idx_dst0, idx_src1, idx_dst1, rows0, rows1,
              semg, sems, nla)
    drain_dumps(2 * (nla + nlb))
    nchunks_a = glob_a + nla
    nchunks_b = glob_b + nlb

    cnt16[pl.ds(0, 16)] = jnp.full((16,), nchunks_a * CG, jnp.int32)
    pltpu.sync_copy(cnt16, cnt_hbm.at[pl.ds(seg_a * 16, 16)])
    cnt16[pl.ds(0, 16)] = jnp.full((16,), nchunks_b * CG, jnp.int32)
    pltpu.sync_copy(cnt16, cnt_hbm.at[pl.ds(seg_b * 16, 16)])

    plsc.subcore_barrier()
    pltpu.sync_copy(zsh.at[pl.ds(s * rpt, rpt)],
                    out_hbm.at[pl.ds((k_a - 1) * n_pad + s * rpt, rpt)])
    plsc.subcore_barrier()

    pltpu.sync_copy(zeros_hbm, zsh.at[pl.ds(s * rpt, rpt)])
    plsc.subcore_barrier()
    _list_gs(h_hbm, zsh, lsrc_hbm, ldst_hbm, ca_src, ca_dst,
             idx_src0, idx_dst0, idx_src1, idx_dst1, rows0, rows1,
             semg, sems, semst, seg_b, nchunks_b)
    plsc.subcore_barrier()
    pltpu.sync_copy(zsh.at[pl.ds(s * rpt, rpt)],
                    out_hbm.at[pl.ds((k_b - 1) * n_pad + s * rpt, rpt)])


def _sc_passn_body(n_k, n_pad,
                   h_hbm, lsrc_hbm, ldst_hbm, cnt_hbm, zeros_hbm, out_hbm,
                   ca_src, ca_dst,
                   idx_src0, idx_dst0, idx_src1, idx_dst1, rows0, rows1,
                   cnt16, zsh, semg, sems, semst):
    c = lax.axis_index("c")
    s = lax.axis_index("s")
    rpt = n_pad // NS
    n_groups = (n_k + 1) // 2

    for g in range(n_groups):
        k = 2 * g + c + 1
        active = k <= n_k
        seg = (k - 1) * NS + s

        @pl.when(active)
        def _zero():
            pltpu.sync_copy(zeros_hbm, zsh.at[pl.ds(s * rpt, rpt)])

        plsc.subcore_barrier()

        @pl.when(active)
        def _work():
            pltpu.sync_copy(cnt_hbm.at[pl.ds(seg * 16, 16)], cnt16)
            v = cnt16[pl.ds(0, 16)]
            nchunks = lax.div(v[0], CG)
            _list_gs(h_hbm, zsh, lsrc_hbm, ldst_hbm, ca_src, ca_dst,
                     idx_src0, idx_dst0, idx_src1, idx_dst1, rows0, rows1,
                     semg, sems, semst, seg, nchunks)

        plsc.subcore_barrier()

        @pl.when(active)
        def _dump():
            pltpu.sync_copy(zsh.at[pl.ds(s * rpt, rpt)],
                            out_hbm.at[pl.ds((k - 1) * n_pad + s * rpt, rpt)])

        plsc.subcore_barrier()


@functools.lru_cache(maxsize=None)
def _make_sc_pass0(n_pad, e_total, n_nodes, d, num_layers):
    mesh = plsc.VectorSubcoreMesh(core_axis_name="c", subcore_axis_name="s")
    comp_cap = CS + 2 * CG + 16
    return pl.kernel(
        functools.partial(_sc_pass0_body, n_pad, e_total, n_nodes),
        out_type=(
            jax.ShapeDtypeStruct((num_layers * n_pad, d), jnp.float32),
            jax.ShapeDtypeStruct((num_layers * NS * CAP,), jnp.int32),
            jax.ShapeDtypeStruct((num_layers * NS * CAP,), jnp.int32),
            jax.ShapeDtypeStruct((num_layers * NS * 16,), jnp.int32),
        ),
        mesh=mesh,
        compiler_params=pltpu.CompilerParams(needs_layout_passes=False),
        scratch_types=[
            pltpu.VMEM((CS,), jnp.int32),
            pltpu.VMEM((CS,), jnp.int32),
            pltpu.VMEM((CS,), jnp.int32),
            pltpu.VMEM((comp_cap,), jnp.int32),
            pltpu.VMEM((comp_cap,), jnp.int32),
            pltpu.VMEM((comp_cap,), jnp.int32),
            pltpu.VMEM((comp_cap,), jnp.int32),
            pltpu.VMEM((CG,), jnp.int32),
            pltpu.VMEM((CG,), jnp.int32),
            pltpu.VMEM((CG,), jnp.int32),
            pltpu.VMEM((CG,), jnp.int32),
            pltpu.VMEM((CG, d), jnp.float32),
            pltpu.VMEM((CG, d), jnp.float32),
            pltpu.VMEM((16,), jnp.int32),
            pltpu.VMEM_SHARED((n_pad, d), jnp.float32),
            pltpu.SemaphoreType.DMA,
            pltpu.SemaphoreType.DMA,
            pltpu.SemaphoreType.DMA,
            pltpu.SemaphoreType.DMA,
        ],
    )


@functools.lru_cache(maxsize=None)
def _make_sc_passn(n_k, n_pad, d):
    mesh = plsc.VectorSubcoreMesh(core_axis_name="c", subcore_axis_name="s")
    comp_cap = CS + 2 * CG + 16
    return pl.kernel(
        functools.partial(_sc_passn_body, n_k, n_pad),
        out_type=jax.ShapeDtypeStruct((n_k * n_pad, d), jnp.float32),
        mesh=mesh,
        compiler_params=pltpu.CompilerParams(needs_layout_passes=False),
        scratch_types=[
            pltpu.VMEM((comp_cap,), jnp.int32),
            pltpu.VMEM((comp_cap,), jnp.int32),
            pltpu.VMEM((CG,), jnp.int32),
            pltpu.VMEM((CG,), jnp.int32),
            pltpu.VMEM((CG,), jnp.int32),
            pltpu.VMEM((CG,), jnp.int32),
            pltpu.VMEM((CG, d), jnp.float32),
            pltpu.VMEM((CG, d), jnp.float32),
            pltpu.VMEM((16,), jnp.int32),
            pltpu.VMEM_SHARED((n_pad, d), jnp.float32),
            pltpu.SemaphoreType.DMA,
            pltpu.SemaphoreType.DMA,
            pltpu.SemaphoreType.DMA,
        ],
    )


def _tc_layer_body(t, alpha_ref, w_ref, b_ref, h_ref, *rest):
    z_refs = rest[:-1]
    out_ref = rest[-1]
    arow = alpha_ref[...]
    col = lax.broadcasted_iota(jnp.int32, arow.shape, 1)
    valid = col < (t + 1)
    masked = jnp.where(valid, arow, -1e30)
    mx = jnp.max(masked, axis=1, keepdims=True)
    ex = jnp.where(valid, jnp.exp(arow - mx), 0.0)
    denom = jnp.sum(ex, axis=1, keepdims=True)
    S = jnp.zeros_like(h_ref[...])
    for i in range(t + 1):
        wi = ex[0:1, i:i + 1] / denom
        S = S + wi * z_refs[i][...]
    acc = jnp.dot(S, w_ref[...], preferred_element_type=jnp.float32)
    acc = acc + b_ref[...]
    hn = h_ref[...] + jnp.maximum(acc, 0.0)
    nrm = jnp.sqrt(jnp.sum(hn * hn, axis=1, keepdims=True)) + 1e-12
    out_ref[...] = hn / nrm


@functools.lru_cache(maxsize=None)
def _make_tc_layer(t, n_nodes, d, n_alpha, bn):
    grid = (n_nodes // bn,)
    in_specs = [
        pl.BlockSpec((1, n_alpha), lambda i: (0, 0)),
        pl.BlockSpec((d, d), lambda i: (0, 0)),
        pl.BlockSpec((1, d), lambda i: (0, 0)),
        pl.BlockSpec((bn, d), lambda i: (i, 0)),
    ] + [pl.BlockSpec((bn, d), lambda i: (i, 0))] * (t + 1)
    return pl.pallas_call(
        functools.partial(_tc_layer_body, t),
        grid=grid,
        in_specs=in_specs,
        out_specs=pl.BlockSpec((bn, d), lambda i: (i, 0)),
        out_shape=jax.ShapeDtypeStruct((n_nodes, d), jnp.float32),
    )


def kernel(x, edge_index, edge_attr, W, b, alpha):
    n_nodes, d = x.shape
    num_layers = W.shape[0]
    e_total = edge_index.shape[1]
    n_pad = ((n_nodes + NS * 8 - 1) // (NS * 8)) * (NS * 8)
    assert num_layers == 4 and e_total % (NS * CS) == 0

    src = edge_index[0].astype(jnp.int32)
    dst = edge_index[1].astype(jnp.int32)
    attr = edge_attr.astype(jnp.int32)
    zeros_blk = jnp.zeros((n_pad // NS, d), jnp.float32)
    alpha = alpha.astype(jnp.float32)

    h = x.astype(jnp.float32)
    Zs = []
    lsrc = ldst = cnt = None
    for t in range(num_layers):
        n_k = num_layers - t
        if t == 0:
            zflat, lsrc, ldst, cnt = _make_sc_pass0(
                n_pad, e_total, n_nodes, d, num_layers)(
                    h, src, dst, attr, zeros_blk)
        else:
            zflat = _make_sc_passn(n_k, n_pad, d)(
                h, lsrc, ldst, cnt, zeros_blk)
        Zs.append(zflat.reshape(n_k, n_pad, d))
        zlist = [Zs[t - k + 1][k - 1, :n_nodes] for k in range(1, t + 2)]
        h = _make_tc_layer(t, n_nodes, d, alpha.shape[1], 1000)(
            alpha[t:t + 1], W[t], b[t].reshape(1, d), h, *zlist)
    return h


# speedup vs baseline: 11.6284x; 1.0559x over previous
import functools

import jax
import jax.numpy as jnp
from jax import lax
from jax.experimental import pallas as pl
from jax.experimental.pallas import tpu as pltpu
from jax.experimental.pallas import tpu_sc as plsc

NS = 16
CS = 2000
CG = 128
SB = 16
CAP = 20480

_PASS_ROUNDS = {
    3: (("full", 1, 0, 0, 2, 16, 1),
        ("half", 3, 32, 2, 3, 40, 3)),
    2: (("full", 1, 0, 0, 2, 16, 1),),
    1: (("half", 1, 0, 0, 1, 8, 1),),
}
_PASS_BUFS = {
    4: (1, 2, 3, 4),
    3: (1, 2, 3, 3),
    2: (1, 2),
    1: (1, 1),
}


def _gs_range(h_hbm, zsh, csrc, cdst, idx_dst0, idx_dst1, rows0, rows1,
              semg, sems, nfull):
    def fill_dst(bufd, off):
        for j in range(CG // 16):
            bufd[pl.ds(j * 16, 16)] = cdst[pl.ds(off + j * 16, 16)]

    def pair_body(p, carry):
        i0 = 2 * p
        both = i0 + 1 < nfull
        pltpu.async_copy(h_hbm.at[csrc.at[pl.ds(i0 * CG, CG)]], rows0, semg)
        fill_dst(idx_dst0, i0 * CG)

        @pl.when(both)
        def _():
            pltpu.async_copy(h_hbm.at[csrc.at[pl.ds((i0 + 1) * CG, CG)]],
                             rows1, semg)
            fill_dst(idx_dst1, (i0 + 1) * CG)

        pltpu.make_async_copy(h_hbm.at[csrc.at[pl.ds(i0 * CG, CG)]],
                              rows0, semg).wait()

        @pl.when(both)
        def _():
            pltpu.make_async_copy(h_hbm.at[csrc.at[pl.ds((i0 + 1) * CG, CG)]],
                                  rows1, semg).wait()

        pltpu.async_copy(rows0, zsh.at[idx_dst0], sems, add=True)

        @pl.when(both)
        def _():
            pltpu.async_copy(rows1, zsh.at[idx_dst1], sems, add=True)

        pltpu.make_async_copy(rows0, zsh.at[idx_dst0], sems).wait()

        @pl.when(both)
        def _():
            pltpu.make_async_copy(rows1, zsh.at[idx_dst1], sems).wait()

        return carry

    lax.fori_loop(0, lax.div(nfull + 1, 2), pair_body, jnp.int32(0))


def _list_gs(h_hbm, zsh, lsrc_hbm, ldst_hbm, ca_src, ca_dst,
             idx_dst0, idx_dst1, rows0, rows1,
             semg, sems, semst, seg, clo, nchunks):
    def sbody(si, carry):
        off = seg * CAP + (clo + si * SB) * CG
        pltpu.async_copy(lsrc_hbm.at[pl.ds(off, SB * CG)],
                         ca_src.at[pl.ds(0, SB * CG)], semst)
        pltpu.async_copy(ldst_hbm.at[pl.ds(off, SB * CG)],
                         ca_dst.at[pl.ds(0, SB * CG)], semst)
        pltpu.make_async_copy(lsrc_hbm.at[pl.ds(off, SB * CG)],
                              ca_src.at[pl.ds(0, SB * CG)], semst).wait()
        pltpu.make_async_copy(ldst_hbm.at[pl.ds(off, SB * CG)],
                              ca_dst.at[pl.ds(0, SB * CG)], semst).wait()
        _gs_range(h_hbm, zsh, ca_src, ca_dst, idx_dst0, idx_dst1,
                  rows0, rows1, semg, sems,
                  jnp.minimum(nchunks - si * SB, SB))
        return carry

    lax.fori_loop(0, lax.div(nchunks + SB - 1, SB), sbody, jnp.int32(0))


def _sc_pass0_body(n_pad, e_total, n_nodes,
                   h_hbm, src_hbm, dst_hbm, attr_hbm, zeros_hbm,
                   out_hbm, lsrc_hbm, ldst_hbm, cnt_hbm,
                   stage_src, stage_dst, stage_attr,
                   ca_src, ca_dst, cb_src, cb_dst,
                   idx_dst0, idx_dst1, rows0, rows1,
                   cnt16, zsh, semg, sems, semd, semst):
    c = lax.axis_index("c")
    s = lax.axis_index("s")
    epw = e_total // NS
    rpt = n_pad // NS
    base_e = s * epw
    k_a = c + 1
    k_b = c + 3
    seg_a = (k_a - 1) * NS + s
    seg_b = (k_b - 1) * NS + s

    def dump_chunks(csrc, cdst, seg, glob, nfull):
        def dbody(i, carry):
            o = seg * CAP + (glob + i) * CG
            pltpu.async_copy(csrc.at[pl.ds(i * CG, CG)], lsrc_hbm.at[pl.ds(o, CG)], semd)
            pltpu.async_copy(cdst.at[pl.ds(i * CG, CG)], ldst_hbm.at[pl.ds(o, CG)], semd)
            return carry

        lax.fori_loop(0, nfull, dbody, jnp.int32(0))

    def drain_dumps(n2):
        def dbody(i, carry):
            pltpu.make_async_copy(ca_src.at[pl.ds(0, CG)],
                                  lsrc_hbm.at[pl.ds(0, CG)], semd).wait()
            return carry

        lax.fori_loop(0, n2, dbody, jnp.int32(0))

    def slide(csrc, cdst, base):
        for j in range(CG // 16):
            t_s = csrc[pl.ds(base + j * 16, 16)]
            t_d = cdst[pl.ds(base + j * 16, 16)]
            csrc[pl.ds(j * 16, 16)] = t_s
            cdst[pl.ds(j * 16, 16)] = t_d

    def pad_tail(csrc, cdst, tail):
        iota16 = lax.iota(jnp.int32, 16)
        zero16 = jnp.zeros((16,), jnp.int32)
        dummy16 = jnp.full((16,), n_nodes, jnp.int32)
        for j in range(CG // 16):
            pos = tail + j * 16 + iota16
            plsc.store_scatter(csrc, [pos], zero16)
            plsc.store_scatter(cdst, [pos], dummy16)

    pltpu.sync_copy(zeros_hbm, zsh.at[pl.ds(s * rpt, rpt)])
    plsc.subcore_barrier()

    def chunk_body(ci, carry):
        tail_a, tail_b, glob_a, glob_b = carry
        off = base_e + ci * CS
        pltpu.async_copy(src_hbm.at[pl.ds(off, CS)], stage_src, semst)
        pltpu.async_copy(dst_hbm.at[pl.ds(off, CS)], stage_dst, semst)
        pltpu.async_copy(attr_hbm.at[pl.ds(off, CS)], stage_attr, semst)
        pltpu.make_async_copy(src_hbm.at[pl.ds(off, CS)], stage_src, semst).wait()
        pltpu.make_async_copy(dst_hbm.at[pl.ds(off, CS)], stage_dst, semst).wait()
        pltpu.make_async_copy(attr_hbm.at[pl.ds(off, CS)], stage_attr, semst).wait()

        def vec_body(v, carry2):
            na, nb = carry2
            a16 = stage_attr[pl.ds(v * 16, 16)]
            s16 = stage_src[pl.ds(v * 16, 16)]
            d16 = stage_dst[pl.ds(v * 16, 16)]
            ma = a16 == k_a
            mb = a16 == k_b
            posa = na + plsc.cumsum(ma.astype(jnp.int32)) - 1
            posb = nb + plsc.cumsum(mb.astype(jnp.int32)) - 1
            plsc.store_scatter(ca_src, [posa], s16, mask=ma)
            plsc.store_scatter(ca_dst, [posa], d16, mask=ma)
            plsc.store_scatter(cb_src, [posb], s16, mask=mb)
            plsc.store_scatter(cb_dst, [posb], d16, mask=mb)
            return posa[15] + 1, posb[15] + 1

        tail_a, tail_b = lax.fori_loop(0, CS // 16, vec_body, (tail_a, tail_b))

        nfa = lax.div(tail_a, CG)
        nfb = lax.div(tail_b, CG)
        dump_chunks(ca_src, ca_dst, seg_a, glob_a, nfa)
        dump_chunks(cb_src, cb_dst, seg_b, glob_b, nfb)
        _gs_range(h_hbm, zsh, ca_src, ca_dst, idx_dst0, idx_dst1,
                  rows0, rows1, semg, sems, nfa)
        drain_dumps(2 * (nfa + nfb))
        slide(ca_src, ca_dst, nfa * CG)
        slide(cb_src, cb_dst, nfb * CG)
        return (tail_a - nfa * CG, tail_b - nfb * CG, glob_a + nfa, glob_b + nfb)

    z32 = jnp.int32(0)
    tail_a, tail_b, glob_a, glob_b = lax.fori_loop(
        0, epw // CS, chunk_body, (z32, z32, z32, z32))

    pad_tail(ca_src, ca_dst, tail_a)
    pad_tail(cb_src, cb_dst, tail_b)
    nla = lax.div(tail_a + CG - 1, CG)
    nlb = lax.div(tail_b + CG - 1, CG)
    dump_chunks(ca_src, ca_dst, seg_a, glob_a, nla)
    dump_chunks(cb_src, cb_dst, seg_b, glob_b, nlb)
    _gs_range(h_hbm, zsh, ca_src, ca_dst, idx_dst0, idx_dst1,
              rows0, rows1, semg, sems, nla)
    drain_dumps(2 * (nla + nlb))
    nchunks_a = glob_a + nla
    nchunks_b = glob_b + nlb

    cnt16[pl.ds(0, 16)] = jnp.full((16,), nchunks_a * CG, jnp.int32)
    pltpu.sync_copy(cnt16, cnt_hbm.at[pl.ds(seg_a * 16, 16)])
    cnt16[pl.ds(0, 16)] = jnp.full((16,), nchunks_b * CG, jnp.int32)
    pltpu.sync_copy(cnt16, cnt_hbm.at[pl.ds(seg_b * 16, 16)])

    plsc.subcore_barrier()
    pltpu.sync_copy(zsh.at[pl.ds(s * rpt, rpt)],
                    out_hbm.at[pl.ds((k_a - 1) * n_pad + s * rpt, rpt)])
    plsc.subcore_barrier()

    pltpu.sync_copy(zeros_hbm, zsh.at[pl.ds(s * rpt, rpt)])
    plsc.subcore_barrier()
    _list_gs(h_hbm, zsh, lsrc_hbm, ldst_hbm, ca_src, ca_dst,
             idx_dst0, idx_dst1, rows0, rows1,
             semg, sems, semst, seg_b, jnp.int32(0), nchunks_b)
    plsc.subcore_barrier()
    pltpu.sync_copy(zsh.at[pl.ds(s * rpt, rpt)],
                    out_hbm.at[pl.ds((k_b - 1) * n_pad + s * rpt, rpt)])


def _sc_passn_body(n_k, n_pad,
                   h_hbm, lsrc_hbm, ldst_hbm, cnt_hbm, zeros_hbm, out_hbm,
                   ca_src, ca_dst, idx_dst0, idx_dst1, rows0, rows1,
                   cnt16, zsh, semg, sems, semst):
    c = lax.axis_index("c")
    s = lax.axis_index("s")
    rpt = n_pad // NS

    for mode, kA, baseA, slotA, kB, baseB, slotB in _PASS_ROUNDS[n_k]:
        segbase = jnp.where(c == 0, baseA, baseB)
        slot = jnp.where(c == 0, slotA, slotB)
        if mode == "full":
            seg = segbase + s
        else:
            seg = segbase + lax.div(s, 2)
            part = s - 2 * lax.div(s, 2)

        pltpu.sync_copy(zeros_hbm, zsh.at[pl.ds(s * rpt, rpt)])
        plsc.subcore_barrier()

        pltpu.sync_copy(cnt_hbm.at[pl.ds(seg * 16, 16)], cnt16)
        v = cnt16[pl.ds(0, 16)]
        nch = lax.div(v[0], CG)
        if mode == "full":
            clo = jnp.int32(0)
            nchunks = nch
        else:
            clo = lax.div(part * nch, 2)
            nchunks = lax.div((part + 1) * nch, 2) - clo
        _list_gs(h_hbm, zsh, lsrc_hbm, ldst_hbm, ca_src, ca_dst,
                 idx_dst0, idx_dst1, rows0, rows1,
                 semg, sems, semst, seg, clo, nchunks)

        plsc.subcore_barrier()
        pltpu.sync_copy(zsh.at[pl.ds(s * rpt, rpt)],
                        out_hbm.at[pl.ds(slot * n_pad + s * rpt, rpt)])
        plsc.subcore_barrier()


@functools.lru_cache(maxsize=None)
def _make_sc_pass0(n_pad, e_total, n_nodes, d, num_layers):
    mesh = plsc.VectorSubcoreMesh(core_axis_name="c", subcore_axis_name="s")
    comp_cap = CS + 2 * CG + 16
    return pl.kernel(
        functools.partial(_sc_pass0_body, n_pad, e_total, n_nodes),
        out_type=(
            jax.ShapeDtypeStruct((num_layers * n_pad, d), jnp.float32),
            jax.ShapeDtypeStruct((num_layers * NS * CAP,), jnp.int32),
            jax.ShapeDtypeStruct((num_layers * NS * CAP,), jnp.int32),
            jax.ShapeDtypeStruct((num_layers * NS * 16,), jnp.int32),
        ),
        mesh=mesh,
        compiler_params=pltpu.CompilerParams(needs_layout_passes=False),
        scratch_types=[
            pltpu.VMEM((CS,), jnp.int32),
            pltpu.VMEM((CS,), jnp.int32),
            pltpu.VMEM((CS,), jnp.int32),
            pltpu.VMEM((comp_cap,), jnp.int32),
            pltpu.VMEM((comp_cap,), jnp.int32),
            pltpu.VMEM((comp_cap,), jnp.int32),
            pltpu.VMEM((comp_cap,), jnp.int32),
            pltpu.VMEM((CG,), jnp.int32),
            pltpu.VMEM((CG,), jnp.int32),
            pltpu.VMEM((CG, d), jnp.float32),
            pltpu.VMEM((CG, d), jnp.float32),
            pltpu.VMEM((16,), jnp.int32),
            pltpu.VMEM_SHARED((n_pad, d), jnp.float32),
            pltpu.SemaphoreType.DMA,
            pltpu.SemaphoreType.DMA,
            pltpu.SemaphoreType.DMA,
            pltpu.SemaphoreType.DMA,
        ],
    )


@functools.lru_cache(maxsize=None)
def _make_sc_passn(n_k, n_pad, d):
    mesh = plsc.VectorSubcoreMesh(core_axis_name="c", subcore_axis_name="s")
    comp_cap = CS + 2 * CG + 16
    n_bufs = len(_PASS_BUFS[n_k])
    return pl.kernel(
        functools.partial(_sc_passn_body, n_k, n_pad),
        out_type=jax.ShapeDtypeStruct((n_bufs * n_pad, d), jnp.float32),
        mesh=mesh,
        compiler_params=pltpu.CompilerParams(needs_layout_passes=False),
        scratch_types=[
            pltpu.VMEM((comp_cap,), jnp.int32),
            pltpu.VMEM((comp_cap,), jnp.int32),
            pltpu.VMEM((CG,), jnp.int32),
            pltpu.VMEM((CG,), jnp.int32),
            pltpu.VMEM((CG, d), jnp.float32),
            pltpu.VMEM((CG, d), jnp.float32),
            pltpu.VMEM((16,), jnp.int32),
            pltpu.VMEM_SHARED((n_pad, d), jnp.float32),
            pltpu.SemaphoreType.DMA,
            pltpu.SemaphoreType.DMA,
            pltpu.SemaphoreType.DMA,
        ],
    )


def _tc_layer_body(t, ks, alpha_ref, w_ref, b_ref, h_ref, *rest):
    z_refs = rest[:-1]
    out_ref = rest[-1]
    arow = alpha_ref[...]
    col = lax.broadcasted_iota(jnp.int32, arow.shape, 1)
    valid = col < (t + 1)
    masked = jnp.where(valid, arow, -1e30)
    mx = jnp.max(masked, axis=1, keepdims=True)
    ex = jnp.where(valid, jnp.exp(arow - mx), 0.0)
    denom = jnp.sum(ex, axis=1, keepdims=True)
    S = jnp.zeros_like(h_ref[...])
    for i, k in enumerate(ks):
        wi = ex[0:1, k - 1:k] / denom
        S = S + wi * z_refs[i][...]
    acc = jnp.dot(S, w_ref[...], preferred_element_type=jnp.float32)
    acc = acc + b_ref[...]
    hn = h_ref[...] + jnp.maximum(acc, 0.0)
    nrm = jnp.sqrt(jnp.sum(hn * hn, axis=1, keepdims=True)) + 1e-12
    out_ref[...] = hn / nrm


@functools.lru_cache(maxsize=None)
def _make_tc_layer(t, ks, n_nodes, d, n_alpha, bn):
    grid = (n_nodes // bn,)
    in_specs = [
        pl.BlockSpec((1, n_alpha), lambda i: (0, 0)),
        pl.BlockSpec((d, d), lambda i: (0, 0)),
        pl.BlockSpec((1, d), lambda i: (0, 0)),
        pl.BlockSpec((bn, d), lambda i: (i, 0)),
    ] + [pl.BlockSpec((bn, d), lambda i: (i, 0))] * len(ks)
    return pl.pallas_call(
        functools.partial(_tc_layer_body, t, ks),
        grid=grid,
        in_specs=in_specs,
        out_specs=pl.BlockSpec((bn, d), lambda i: (i, 0)),
        out_shape=jax.ShapeDtypeStruct((n_nodes, d), jnp.float32),
    )


def kernel(x, edge_index, edge_attr, W, b, alpha):
    n_nodes, d = x.shape
    num_layers = W.shape[0]
    e_total = edge_index.shape[1]
    n_pad = ((n_nodes + NS * 8 - 1) // (NS * 8)) * (NS * 8)
    assert num_layers == 4 and e_total % (NS * CS) == 0

    src = edge_index[0].astype(jnp.int32)
    dst = edge_index[1].astype(jnp.int32)
    attr = edge_attr.astype(jnp.int32)
    zeros_blk = jnp.zeros((n_pad // NS, d), jnp.float32)
    alpha = alpha.astype(jnp.float32)

    h = x.astype(jnp.float32)
    pass_bufs = []
    lsrc = ldst = cnt = None
    for t in range(num_layers):
        n_k = num_layers - t
        if t == 0:
            zflat, lsrc, ldst, cnt = _make_sc_pass0(
                n_pad, e_total, n_nodes, d, num_layers)(
                    h, src, dst, attr, zeros_blk)
        else:
            zflat = _make_sc_passn(n_k, n_pad, d)(
                h, lsrc, ldst, cnt, zeros_blk)
        n_bufs = len(_PASS_BUFS[n_k])
        zstack = zflat.reshape(n_bufs, n_pad, d)
        pass_bufs.append(list(zip(_PASS_BUFS[n_k],
                                  [zstack[i, :n_nodes] for i in range(n_bufs)])))
        zlist, ks = [], []
        for k in range(1, t + 2):
            for bk, buf in pass_bufs[t - k + 1]:
                if bk == k:
                    zlist.append(buf)
                    ks.append(k)
        h = _make_tc_layer(t, tuple(ks), n_nodes, d, alpha.shape[1], 1000)(
            alpha[t:t + 1], W[t], b[t].reshape(1, d), h, *zlist)
    return h


# speedup vs baseline: 12.1082x; 1.0413x over previous
import functools

import jax
import jax.numpy as jnp
from jax import lax
from jax.experimental import pallas as pl
from jax.experimental.pallas import tpu as pltpu
from jax.experimental.pallas import tpu_sc as plsc

NS = 16
CS = 2000
CG = 128
SB = 16
CAP = 20480

_PASS_ROUNDS = {
    3: (("full", 1, 0, 0, 2, 16, 1),
        ("half", 3, 32, 2, 3, 40, 3)),
    2: (("full", 1, 0, 0, 2, 16, 1),),
    1: (("half", 1, 0, 0, 1, 8, 1),),
}
_PASS_BUFS = {
    4: (1, 2, 3, 4),
    3: (1, 2, 3, 3),
    2: (1, 2),
    1: (1, 1),
}


def _gs_range(h_hbm, zsh, csrc, cdst, idx_dst0, idx_dst1, rows0, rows1,
              semg, sems, nfull):
    def fill_dst(bufd, off):
        for j in range(CG // 16):
            bufd[pl.ds(j * 16, 16)] = cdst[pl.ds(off + j * 16, 16)]

    def start_g(i, rows):
        pltpu.async_copy(h_hbm.at[csrc.at[pl.ds(i * CG, CG)]], rows, semg)

    def wait_g(i, rows):
        pltpu.make_async_copy(h_hbm.at[csrc.at[pl.ds(i * CG, CG)]],
                              rows, semg).wait()

    @pl.when(nfull > 0)
    def _():
        start_g(0, rows0)

    def pair_body(p, carry):
        i0 = 2 * p
        fill_dst(idx_dst0, i0 * CG)
        wait_g(i0, rows0)
        pltpu.async_copy(rows0, zsh.at[idx_dst0], sems, add=True)

        @pl.when(i0 + 1 < nfull)
        def _():
            start_g(i0 + 1, rows1)

        pltpu.make_async_copy(rows0, zsh.at[idx_dst0], sems).wait()

        @pl.when(i0 + 1 < nfull)
        def _():
            fill_dst(idx_dst1, (i0 + 1) * CG)
            wait_g(i0 + 1, rows1)
            pltpu.async_copy(rows1, zsh.at[idx_dst1], sems, add=True)

            @pl.when(i0 + 2 < nfull)
            def _():
                start_g(i0 + 2, rows0)

            pltpu.make_async_copy(rows1, zsh.at[idx_dst1], sems).wait()

        return carry

    lax.fori_loop(0, lax.div(nfull + 1, 2), pair_body, jnp.int32(0))


def _list_gs(h_hbm, zsh, lsrc_hbm, ldst_hbm, ca_src, ca_dst,
             idx_dst0, idx_dst1, rows0, rows1,
             semg, sems, semst, seg, clo, nchunks):
    def sbody(si, carry):
        off = seg * CAP + (clo + si * SB) * CG
        pltpu.async_copy(lsrc_hbm.at[pl.ds(off, SB * CG)],
                         ca_src.at[pl.ds(0, SB * CG)], semst)
        pltpu.async_copy(ldst_hbm.at[pl.ds(off, SB * CG)],
                         ca_dst.at[pl.ds(0, SB * CG)], semst)
        pltpu.make_async_copy(lsrc_hbm.at[pl.ds(off, SB * CG)],
                              ca_src.at[pl.ds(0, SB * CG)], semst).wait()
        pltpu.make_async_copy(ldst_hbm.at[pl.ds(off, SB * CG)],
                              ca_dst.at[pl.ds(0, SB * CG)], semst).wait()
        _gs_range(h_hbm, zsh, ca_src, ca_dst, idx_dst0, idx_dst1,
                  rows0, rows1, semg, sems,
                  jnp.minimum(nchunks - si * SB, SB))
        return carry

    lax.fori_loop(0, lax.div(nchunks + SB - 1, SB), sbody, jnp.int32(0))


def _sc_pass0_body(n_pad, e_total, n_nodes,
                   h_hbm, src_hbm, dst_hbm, attr_hbm, zeros_hbm,
                   out_hbm, lsrc_hbm, ldst_hbm, cnt_hbm,
                   stage_src, stage_dst, stage_attr,
                   ca_src, ca_dst, cb_src, cb_dst,
                   idx_dst0, idx_dst1, rows0, rows1,
                   cnt16, zsh, semg, sems, semd, semst):
    c = lax.axis_index("c")
    s = lax.axis_index("s")
    epw = e_total // NS
    rpt = n_pad // NS
    base_e = s * epw
    k_a = c + 1
    k_b = c + 3
    seg_a = (k_a - 1) * NS + s
    seg_b = (k_b - 1) * NS + s

    def dump_chunks(csrc, cdst, seg, glob, nfull):
        def dbody(i, carry):
            o = seg * CAP + (glob + i) * CG
            pltpu.async_copy(csrc.at[pl.ds(i * CG, CG)], lsrc_hbm.at[pl.ds(o, CG)], semd)
            pltpu.async_copy(cdst.at[pl.ds(i * CG, CG)], ldst_hbm.at[pl.ds(o, CG)], semd)
            return carry

        lax.fori_loop(0, nfull, dbody, jnp.int32(0))

    def drain_dumps(n2):
        def dbody(i, carry):
            pltpu.make_async_copy(ca_src.at[pl.ds(0, CG)],
                                  lsrc_hbm.at[pl.ds(0, CG)], semd).wait()
            return carry

        lax.fori_loop(0, n2, dbody, jnp.int32(0))

    def slide(csrc, cdst, base):
        for j in range(CG // 16):
            t_s = csrc[pl.ds(base + j * 16, 16)]
            t_d = cdst[pl.ds(base + j * 16, 16)]
            csrc[pl.ds(j * 16, 16)] = t_s
            cdst[pl.ds(j * 16, 16)] = t_d

    def pad_tail(csrc, cdst, tail):
        iota16 = lax.iota(jnp.int32, 16)
        zero16 = jnp.zeros((16,), jnp.int32)
        dummy16 = jnp.full((16,), n_nodes, jnp.int32)
        for j in range(CG // 16):
            pos = tail + j * 16 + iota16
            plsc.store_scatter(csrc, [pos], zero16)
            plsc.store_scatter(cdst, [pos], dummy16)

    pltpu.sync_copy(zeros_hbm, zsh.at[pl.ds(s * rpt, rpt)])
    plsc.subcore_barrier()

    def chunk_body(ci, carry):
        tail_a, tail_b, glob_a, glob_b = carry
        off = base_e + ci * CS
        pltpu.async_copy(src_hbm.at[pl.ds(off, CS)], stage_src, semst)
        pltpu.async_copy(dst_hbm.at[pl.ds(off, CS)], stage_dst, semst)
        pltpu.async_copy(attr_hbm.at[pl.ds(off, CS)], stage_attr, semst)
        pltpu.make_async_copy(src_hbm.at[pl.ds(off, CS)], stage_src, semst).wait()
        pltpu.make_async_copy(dst_hbm.at[pl.ds(off, CS)], stage_dst, semst).wait()
        pltpu.make_async_copy(attr_hbm.at[pl.ds(off, CS)], stage_attr, semst).wait()

        def vec_body(v, carry2):
            na, nb = carry2
            a16 = stage_attr[pl.ds(v * 16, 16)]
            s16 = stage_src[pl.ds(v * 16, 16)]
            d16 = stage_dst[pl.ds(v * 16, 16)]
            ma = a16 == k_a
            mb = a16 == k_b
            posa = na + plsc.cumsum(ma.astype(jnp.int32)) - 1
            posb = nb + plsc.cumsum(mb.astype(jnp.int32)) - 1
            plsc.store_scatter(ca_src, [posa], s16, mask=ma)
            plsc.store_scatter(ca_dst, [posa], d16, mask=ma)
            plsc.store_scatter(cb_src, [posb], s16, mask=mb)
            plsc.store_scatter(cb_dst, [posb], d16, mask=mb)
            return posa[15] + 1, posb[15] + 1

        tail_a, tail_b = lax.fori_loop(0, CS // 16, vec_body, (tail_a, tail_b))

        nfa = lax.div(tail_a, CG)
        nfb = lax.div(tail_b, CG)
        dump_chunks(ca_src, ca_dst, seg_a, glob_a, nfa)
        dump_chunks(cb_src, cb_dst, seg_b, glob_b, nfb)
        _gs_range(h_hbm, zsh, ca_src, ca_dst, idx_dst0, idx_dst1,
                  rows0, rows1, semg, sems, nfa)
        drain_dumps(2 * (nfa + nfb))
        slide(ca_src, ca_dst, nfa * CG)
        slide(cb_src, cb_dst, nfb * CG)
        return (tail_a - nfa * CG, tail_b - nfb * CG, glob_a + nfa, glob_b + nfb)

    z32 = jnp.int32(0)
    tail_a, tail_b, glob_a, glob_b = lax.fori_loop(
        0, epw // CS, chunk_body, (z32, z32, z32, z32))

    pad_tail(ca_src, ca_dst, tail_a)
    pad_tail(cb_src, cb_dst, tail_b)
    nla = lax.div(tail_a + CG - 1, CG)
    nlb = lax.div(tail_b + CG - 1, CG)
    dump_chunks(ca_src, ca_dst, seg_a, glob_a, nla)
    dump_chunks(cb_src, cb_dst, seg_b, glob_b, nlb)
    _gs_range(h_hbm, zsh, ca_src, ca_dst, idx_dst0, idx_dst1,
              rows0, rows1, semg, sems, nla)
    drain_dumps(2 * (nla + nlb))
    nchunks_a = glob_a + nla
    nchunks_b = glob_b + nlb

    cnt16[pl.ds(0, 16)] = jnp.full((16,), nchunks_a * CG, jnp.int32)
    pltpu.sync_copy(cnt16, cnt_hbm.at[pl.ds(seg_a * 16, 16)])
    cnt16[pl.ds(0, 16)] = jnp.full((16,), nchunks_b * CG, jnp.int32)
    pltpu.sync_copy(cnt16, cnt_hbm.at[pl.ds(seg_b * 16, 16)])

    plsc.subcore_barrier()
    pltpu.sync_copy(zsh.at[pl.ds(s * rpt, rpt)],
                    out_hbm.at[pl.ds((k_a - 1) * n_pad + s * rpt, rpt)])
    plsc.subcore_barrier()

    pltpu.sync_copy(zeros_hbm, zsh.at[pl.ds(s * rpt, rpt)])
    plsc.subcore_barrier()
    _list_gs(h_hbm, zsh, lsrc_hbm, ldst_hbm, ca_src, ca_dst,
             idx_dst0, idx_dst1, rows0, rows1,
             semg, sems, semst, seg_b, jnp.int32(0), nchunks_b)
    plsc.subcore_barrier()
    pltpu.sync_copy(zsh.at[pl.ds(s * rpt, rpt)],
                    out_hbm.at[pl.ds((k_b - 1) * n_pad + s * rpt, rpt)])


def _sc_passn_body(n_k, n_pad,
                   h_hbm, lsrc_hbm, ldst_hbm, cnt_hbm, zeros_hbm, out_hbm,
                   ca_src, ca_dst, idx_dst0, idx_dst1, rows0, rows1,
                   cnt16, zsh, semg, sems, semst):
    c = lax.axis_index("c")
    s = lax.axis_index("s")
    rpt = n_pad // NS

    for mode, kA, baseA, slotA, kB, baseB, slotB in _PASS_ROUNDS[n_k]:
        segbase = jnp.where(c == 0, baseA, baseB)
        slot = jnp.where(c == 0, slotA, slotB)
        if mode == "full":
            seg = segbase + s
        else:
            seg = segbase + lax.div(s, 2)
            part = s - 2 * lax.div(s, 2)

        pltpu.sync_copy(zeros_hbm, zsh.at[pl.ds(s * rpt, rpt)])
        plsc.subcore_barrier()

        pltpu.sync_copy(cnt_hbm.at[pl.ds(seg * 16, 16)], cnt16)
        v = cnt16[pl.ds(0, 16)]
        nch = lax.div(v[0], CG)
        if mode == "full":
            clo = jnp.int32(0)
            nchunks = nch
        else:
            clo = lax.div(part * nch, 2)
            nchunks = lax.div((part + 1) * nch, 2) - clo
        _list_gs(h_hbm, zsh, lsrc_hbm, ldst_hbm, ca_src, ca_dst,
                 idx_dst0, idx_dst1, rows0, rows1,
                 semg, sems, semst, seg, clo, nchunks)

        plsc.subcore_barrier()
        pltpu.sync_copy(zsh.at[pl.ds(s * rpt, rpt)],
                        out_hbm.at[pl.ds(slot * n_pad + s * rpt, rpt)])
        plsc.subcore_barrier()


@functools.lru_cache(maxsize=None)
def _make_sc_pass0(n_pad, e_total, n_nodes, d, num_layers):
    mesh = plsc.VectorSubcoreMesh(core_axis_name="c", subcore_axis_name="s")
    comp_cap = CS + 2 * CG + 16
    return pl.kernel(
        functools.partial(_sc_pass0_body, n_pad, e_total, n_nodes),
        out_type=(
            jax.ShapeDtypeStruct((num_layers * n_pad, d), jnp.float32),
            jax.ShapeDtypeStruct((num_layers * NS * CAP,), jnp.int32),
            jax.ShapeDtypeStruct((num_layers * NS * CAP,), jnp.int32),
            jax.ShapeDtypeStruct((num_layers * NS * 16,), jnp.int32),
        ),
        mesh=mesh,
        compiler_params=pltpu.CompilerParams(needs_layout_passes=False),
        scratch_types=[
            pltpu.VMEM((CS,), jnp.int32),
            pltpu.VMEM((CS,), jnp.int32),
            pltpu.VMEM((CS,), jnp.int32),
            pltpu.VMEM((comp_cap,), jnp.int32),
            pltpu.VMEM((comp_cap,), jnp.int32),
            pltpu.VMEM((comp_cap,), jnp.int32),
            pltpu.VMEM((comp_cap,), jnp.int32),
            pltpu.VMEM((CG,), jnp.int32),
            pltpu.VMEM((CG,), jnp.int32),
            pltpu.VMEM((CG, d), jnp.float32),
            pltpu.VMEM((CG, d), jnp.float32),
            pltpu.VMEM((16,), jnp.int32),
            pltpu.VMEM_SHARED((n_pad, d), jnp.float32),
            pltpu.SemaphoreType.DMA,
            pltpu.SemaphoreType.DMA,
            pltpu.SemaphoreType.DMA,
            pltpu.SemaphoreType.DMA,
        ],
    )


@functools.lru_cache(maxsize=None)
def _make_sc_passn(n_k, n_pad, d):
    mesh = plsc.VectorSubcoreMesh(core_axis_name="c", subcore_axis_name="s")
    comp_cap = CS + 2 * CG + 16
    n_bufs = len(_PASS_BUFS[n_k])
    return pl.kernel(
        functools.partial(_sc_passn_body, n_k, n_pad),
        out_type=jax.ShapeDtypeStruct((n_bufs * n_pad, d), jnp.float32),
        mesh=mesh,
        compiler_params=pltpu.CompilerParams(needs_layout_passes=False),
        scratch_types=[
            pltpu.VMEM((comp_cap,), jnp.int32),
            pltpu.VMEM((comp_cap,), jnp.int32),
            pltpu.VMEM((CG,), jnp.int32),
            pltpu.VMEM((CG,), jnp.int32),
            pltpu.VMEM((CG, d), jnp.float32),
            pltpu.VMEM((CG, d), jnp.float32),
            pltpu.VMEM((16,), jnp.int32),
            pltpu.VMEM_SHARED((n_pad, d), jnp.float32),
            pltpu.SemaphoreType.DMA,
            pltpu.SemaphoreType.DMA,
            pltpu.SemaphoreType.DMA,
        ],
    )


def _tc_layer_body(t, ks, alpha_ref, w_ref, b_ref, h_ref, *rest):
    z_refs = rest[:-1]
    out_ref = rest[-1]
    arow = alpha_ref[...]
    col = lax.broadcasted_iota(jnp.int32, arow.shape, 1)
    valid = col < (t + 1)
    masked = jnp.where(valid, arow, -1e30)
    mx = jnp.max(masked, axis=1, keepdims=True)
    ex = jnp.where(valid, jnp.exp(arow - mx), 0.0)
    denom = jnp.sum(ex, axis=1, keepdims=True)
    S = jnp.zeros_like(h_ref[...])
    for i, k in enumerate(ks):
        wi = ex[0:1, k - 1:k] / denom
        S = S + wi * z_refs[i][...]
    acc = jnp.dot(S, w_ref[...], preferred_element_type=jnp.float32)
    acc = acc + b_ref[...]
    hn = h_ref[...] + jnp.maximum(acc, 0.0)
    nrm = jnp.sqrt(jnp.sum(hn * hn, axis=1, keepdims=True)) + 1e-12
    out_ref[...] = hn / nrm


@functools.lru_cache(maxsize=None)
def _make_tc_layer(t, ks, n_nodes, d, n_alpha, bn):
    grid = (n_nodes // bn,)
    in_specs = [
        pl.BlockSpec((1, n_alpha), lambda i: (0, 0)),
        pl.BlockSpec((d, d), lambda i: (0, 0)),
        pl.BlockSpec((1, d), lambda i: (0, 0)),
        pl.BlockSpec((bn, d), lambda i: (i, 0)),
    ] + [pl.BlockSpec((bn, d), lambda i: (i, 0))] * len(ks)
    return pl.pallas_call(
        functools.partial(_tc_layer_body, t, ks),
        grid=grid,
        in_specs=in_specs,
        out_specs=pl.BlockSpec((bn, d), lambda i: (i, 0)),
        out_shape=jax.ShapeDtypeStruct((n_nodes, d), jnp.float32),
    )


def kernel(x, edge_index, edge_attr, W, b, alpha):
    n_nodes, d = x.shape
    num_layers = W.shape[0]
    e_total = edge_index.shape[1]
    n_pad = ((n_nodes + NS * 8 - 1) // (NS * 8)) * (NS * 8)
    assert num_layers == 4 and e_total % (NS * CS) == 0

    src = edge_index[0].astype(jnp.int32)
    dst = edge_index[1].astype(jnp.int32)
    attr = edge_attr.astype(jnp.int32)
    zeros_blk = jnp.zeros((n_pad // NS, d), jnp.float32)
    alpha = alpha.astype(jnp.float32)

    h = x.astype(jnp.float32)
    pass_bufs = []
    lsrc = ldst = cnt = None
    for t in range(num_layers):
        n_k = num_layers - t
        if t == 0:
            zflat, lsrc, ldst, cnt = _make_sc_pass0(
                n_pad, e_total, n_nodes, d, num_layers)(
                    h, src, dst, attr, zeros_blk)
        else:
            zflat = _make_sc_passn(n_k, n_pad, d)(
                h, lsrc, ldst, cnt, zeros_blk)
        n_bufs = len(_PASS_BUFS[n_k])
        zstack = zflat.reshape(n_bufs, n_pad, d)
        pass_bufs.append(list(zip(_PASS_BUFS[n_k],
                                  [zstack[i, :n_nodes] for i in range(n_bufs)])))
        zlist, ks = [], []
        for k in range(1, t + 2):
            for bk, buf in pass_bufs[t - k + 1]:
                if bk == k:
                    zlist.append(buf)
                    ks.append(k)
        h = _make_tc_layer(t, tuple(ks), n_nodes, d, alpha.shape[1], 1000)(
            alpha[t:t + 1], W[t], b[t].reshape(1, d), h, *zlist)
    return h
